# baseline (speedup 1.0000x reference)
import functools

import jax
import jax.numpy as jnp
from jax import lax
from jax.experimental import pallas as pl
from jax.experimental.pallas import tpu as pltpu
from jax.experimental.pallas import tpu_sc as plsc

N = 10000
D = 128
E = 50000
NTYPES = 7
NREL = 12
NP = 10240
NC = 2
NS = 16
G = 33
GW = 96
EP = NS * G * GW
BLK = 1024
NBLK = NP // BLK

SRC_T = [0, 0, 0, 0, 0, 0, 1, 2, 3, 4, 5, 6]
DST_T = [1, 2, 3, 4, 5, 6, 0, 0, 0, 0, 0, 0]


def _mm_body(x0, x1, x2, x3, x4, x5, x6, wy_ref, wr_ref, y_ref, z_ref):
    xs = [x0[...], x1[...], x2[...], x3[...], x4[...], x5[...], x6[...]]
    for r in range(NREL):
        y_ref[r] = jnp.dot(xs[SRC_T[r]], wy_ref[r],
                           preferred_element_type=jnp.float32)
    for t in range(NTYPES):
        z_ref[t] = jnp.dot(xs[t], wr_ref[t], preferred_element_type=jnp.float32)


def _mm_stage(xs7, wy, wr_eff):
    blk_x = pl.BlockSpec((BLK, D), lambda i: (i, 0))
    return pl.pallas_call(
        _mm_body,
        grid=(NBLK,),
        in_specs=[blk_x] * 7 + [
            pl.BlockSpec((NREL, D, D), lambda i: (0, 0, 0)),
            pl.BlockSpec((NTYPES, D, D), lambda i: (0, 0, 0)),
        ],
        out_specs=[
            pl.BlockSpec((NREL, BLK, D), lambda i: (0, i, 0)),
            pl.BlockSpec((NTYPES, BLK, D), lambda i: (0, i, 0)),
        ],
        out_shape=[
            jax.ShapeDtypeStruct((NREL, NP, D), jnp.float32),
            jax.ShapeDtypeStruct((NTYPES, NP, D), jnp.float32),
        ],
    )(*xs7, wy, wr_eff)


ROWS_PER_TILE = NP // NS


NB = 3
AHEAD = 2


def _sc_body(y_hbm, src_hbm, dst_hbm, zeros_hbm, agg_hbm, acc, srcv, dstv,
             rows, zsem, *gsem):
    c = lax.axis_index("c")
    s = lax.axis_index("s")

    def one_rel(rr, carry):
        r = c * 6 + rr
        zd = pltpu.async_copy(
            zeros_hbm, acc.at[pl.ds(s * ROWS_PER_TILE, ROWS_PER_TILE)], zsem)
        pltpu.sync_copy(src_hbm.at[r, s], srcv)
        pltpu.sync_copy(dst_hbm.at[r, s], dstv)
        gd = [None] * G
        for g in range(AHEAD):
            gd[g] = pltpu.async_copy(y_hbm.at[r].at[srcv.at[g]],
                                     rows.at[g % NB], gsem[g % NB])
        zd.wait()
        plsc.subcore_barrier()
        for g in range(G):
            b = g % NB
            gd[g].wait()
            gn = g + AHEAD
            if gn < G:
                gd[gn] = pltpu.async_copy(y_hbm.at[r].at[srcv.at[gn]],
                                          rows.at[gn % NB], gsem[gn % NB])
            pltpu.sync_copy(rows.at[b], acc.at[dstv.at[g]], add=True)
        plsc.subcore_barrier()
        pltpu.sync_copy(acc.at[pl.ds(s * ROWS_PER_TILE, ROWS_PER_TILE)],
                        agg_hbm.at[r, pl.ds(s * ROWS_PER_TILE, ROWS_PER_TILE)])
        return carry

    lax.fori_loop(0, 6, one_rel, 0)


def _sc_segment(y, src_idx, dst_idx, zeros_tile):
    mesh = plsc.VectorSubcoreMesh(core_axis_name="c", subcore_axis_name="s")
    f = pl.kernel(
        _sc_body,
        out_type=jax.ShapeDtypeStruct((NREL, NP, D), jnp.float32),
        mesh=mesh,
        scratch_types=[
            pltpu.VMEM_SHARED((NP, D), jnp.float32),
            pltpu.VMEM((G, GW), jnp.int32),
            pltpu.VMEM((G, GW), jnp.int32),
            pltpu.VMEM((NB, GW, D), jnp.float32),
        ] + [pltpu.SemaphoreType.DMA] * (NB + 1),
    )
    return f(y, src_idx, dst_idx, zeros_tile)


GC = 25
EPC = NS * GC * 128
CW = 8
FLATN = NP * CW


def _sc_count_body(flat_hbm, zeros_hbm, cnt_hbm, acc, flatv, onesv, zsem):
    c = lax.axis_index("c")
    s = lax.axis_index("s")
    one16 = jnp.ones((16,), jnp.float32)
    for j in range(128 // 16):
        onesv[pl.ds(j * 16, 16)] = one16
    chunk = FLATN // NS
    zd = pltpu.async_copy(zeros_hbm, acc.at[pl.ds(s * chunk, chunk)], zsem)

    def one_rel(rr, carry):
        r = c * 6 + rr
        pltpu.sync_copy(flat_hbm.at[r, s], flatv)
        for g in range(GC):
            pltpu.sync_copy(onesv, acc.at[flatv.at[g]], add=True)
        return carry

    pltpu.sync_copy(flat_hbm.at[c * 6, s], flatv)
    zd.wait()
    plsc.subcore_barrier()
    for g in range(GC):
        pltpu.sync_copy(onesv, acc.at[flatv.at[g]], add=True)
    lax.fori_loop(1, 6, one_rel, 0)
    plsc.subcore_barrier()
    pltpu.sync_copy(acc.at[pl.ds(s * chunk, chunk)],
                    cnt_hbm.at[c, pl.ds(s * chunk, chunk)])


def _sc_count(flat_idx, zeros_flat):
    mesh = plsc.VectorSubcoreMesh(core_axis_name="c", subcore_axis_name="s")
    f = pl.kernel(
        _sc_count_body,
        out_type=jax.ShapeDtypeStruct((NC, FLATN), jnp.float32),
        mesh=mesh,
        scratch_types=[
            pltpu.VMEM_SHARED((FLATN,), jnp.float32),
            pltpu.VMEM((GC, 128), jnp.int32),
            pltpu.VMEM((128,), jnp.float32),
            pltpu.SemaphoreType.DMA,
        ],
    )
    return f(flat_idx, zeros_flat)


def _mean_term(agg_ref, cnt0_ref, cnt1_ref, r):
    cref = cnt0_ref if r < 6 else cnt1_ref
    col = r % 6
    c = jnp.maximum(cref[:, col:col + 1], 1.0)
    return agg_ref[r] / c


FBLK = 512


def _fused_cm_body(agg_ref, cnt0_ref, cnt1_ref, z_ref, b_ref, wy_ref, wr_ref,
                  y_ref, zo_ref, *h_ref):
    i = pl.program_id(0)
    rowid = i * FBLK + lax.broadcasted_iota(jnp.int32, (FBLK, 1), 0)
    mask = (rowid < N).astype(jnp.float32)
    hs = []
    for t in range(NTYPES):
        if t == 0:
            m = _mean_term(agg_ref, cnt0_ref, cnt1_ref, 6)
            for r in range(7, 12):
                m = m + _mean_term(agg_ref, cnt0_ref, cnt1_ref, r)
        else:
            m = _mean_term(agg_ref, cnt0_ref, cnt1_ref, t - 1)
        h = jnp.maximum(m + z_ref[t] + b_ref[t:t + 1, :], 0.0) * mask
        h_ref[t][...] = h
        hs.append(h)
    for r in range(NREL):
        y_ref[r] = jnp.dot(hs[SRC_T[r]], wy_ref[r],
                           preferred_element_type=jnp.float32)
    for t in range(NTYPES):
        zo_ref[t] = jnp.dot(hs[t], wr_ref[t], preferred_element_type=jnp.float32)


def _fused_cm_stage(agg, cnt0, cnt1, z, bias, wy, wr_eff):
    blk_h = pl.BlockSpec((FBLK, D), lambda i: (i, 0))
    blk_rel = pl.BlockSpec((NREL, FBLK, D), lambda i: (0, i, 0))
    blk_typ = pl.BlockSpec((NTYPES, FBLK, D), lambda i: (0, i, 0))
    blk_cnt = pl.BlockSpec((FBLK, CW), lambda i: (i, 0))
    outs = pl.pallas_call(
        _fused_cm_body,
        grid=(NP // FBLK,),
        in_specs=[
            blk_rel, blk_cnt, blk_cnt, blk_typ,
            pl.BlockSpec((NTYPES, D), lambda i: (0, 0)),
            pl.BlockSpec((NREL, D, D), lambda i: (0, 0, 0)),
            pl.BlockSpec((NTYPES, D, D), lambda i: (0, 0, 0)),
        ],
        out_specs=[blk_rel, blk_typ] + [blk_h] * NTYPES,
        out_shape=[
            jax.ShapeDtypeStruct((NREL, NP, D), jnp.float32),
            jax.ShapeDtypeStruct((NTYPES, NP, D), jnp.float32),
        ] + [jax.ShapeDtypeStruct((NP, D), jnp.float32)] * NTYPES,
    )(agg, cnt0, cnt1, z, bias, wy, wr_eff)
    return outs[0], outs[1], outs[2:]


def _combine_body(residual, *refs):
    agg_ref, cnt_ref, z_ref, b_ref = refs[:4]
    h1 = refs[4:4 + NTYPES] if residual else ()
    outs = refs[4 + NTYPES:] if residual else refs[4:]
    i = pl.program_id(0)
    rowid = i * BLK + lax.broadcasted_iota(jnp.int32, (BLK, 1), 0)
    mask = (rowid < N).astype(jnp.float32)
    for t in range(NTYPES):
        if t == 0:
            m = _mean_term(agg_ref, cnt_ref, 6)
            for r in range(7, 12):
                m = m + _mean_term(agg_ref, cnt_ref, r)
        else:
            m = _mean_term(agg_ref, cnt_ref, t - 1)
        h = jnp.maximum(m + z_ref[t] + b_ref[t:t + 1, :], 0.0) * mask
        if residual:
            h = h + h1[t][...]
        outs[t][...] = h


def _combine(agg, cnt, z, bias, h1s=None):
    residual = h1s is not None
    blk_out = pl.BlockSpec((BLK, D), lambda i: (i, 0))
    blk_rel = pl.BlockSpec((NREL, BLK, D), lambda i: (0, i, 0))
    in_specs = [
        blk_rel,
        blk_rel,
        pl.BlockSpec((NTYPES, BLK, D), lambda i: (0, i, 0)),
        pl.BlockSpec((NTYPES, D), lambda i: (0, 0)),
    ]
    args = [agg, cnt, z, bias]
    if residual:
        in_specs += [blk_out] * NTYPES
        args += list(h1s)
    return pl.pallas_call(
        functools.partial(_combine_body, residual),
        grid=(NBLK,),
        in_specs=in_specs,
        out_specs=[blk_out] * NTYPES,
        out_shape=[jax.ShapeDtypeStruct((NP, D), jnp.float32)] * NTYPES,
    )(*args)


def _final_body(agg_ref, cnt0_ref, cnt1_ref, z_ref, b_ref, h1_0, h1_1, h1_2,
                h1_3, h1_4, h1_5, h1_6, l2w_ref, l2b_ref, lw_ref, lb_ref,
                out_ref, acc):
    h1 = [h1_0, h1_1, h1_2, h1_3, h1_4, h1_5, h1_6]
    i = pl.program_id(0)

    @pl.when(i == 0)
    def _():
        acc[...] = jnp.zeros((8, D), jnp.float32)

    rowid = i * BLK + lax.broadcasted_iota(jnp.int32, (BLK, 1), 0)
    mask = (rowid < N).astype(jnp.float32)
    psum = jnp.zeros((1, D), jnp.float32)
    for t in range(NTYPES):
        if t == 0:
            m = _mean_term(agg_ref, cnt0_ref, cnt1_ref, 6)
            for r in range(7, 12):
                m = m + _mean_term(agg_ref, cnt0_ref, cnt1_ref, r)
        else:
            m = _mean_term(agg_ref, cnt0_ref, cnt1_ref, t - 1)
        h = jnp.maximum(m + z_ref[t] + b_ref[t:t + 1, :], 0.0) * mask
        h = h + h1[t][...] * mask
        psum = psum + jnp.sum(h, axis=0, keepdims=True)
    acc[0:1, :] += psum

    @pl.when(i == NBLK - 1)
    def _():
        mvec = acc[0:1, :] * (1.0 / (NTYPES * N))
        o = jnp.dot(mvec, l2w_ref[...], preferred_element_type=jnp.float32)
        o = o + l2b_ref[...]
        o = jnp.dot(o, lw_ref[...], preferred_element_type=jnp.float32)
        out_ref[...] = o + lb_ref[...]


def _final(agg, cnt0, cnt1, z, bias, h1s, lin2_w, lin2_b, lin_w, lin_b):
    blk_h = pl.BlockSpec((BLK, D), lambda i: (i, 0))
    blk_rel = pl.BlockSpec((NREL, BLK, D), lambda i: (0, i, 0))
    blk_cnt = pl.BlockSpec((BLK, CW), lambda i: (i, 0))
    wspec = pl.BlockSpec((D, D), lambda i: (0, 0))
    bspec = pl.BlockSpec((1, D), lambda i: (0, 0))
    return pl.pallas_call(
        _final_body,
        grid=(NBLK,),
        in_specs=[
            blk_rel,
            blk_cnt,
            blk_cnt,
            pl.BlockSpec((NTYPES, BLK, D), lambda i: (0, i, 0)),
            pl.BlockSpec((NTYPES, D), lambda i: (0, 0)),
        ] + [blk_h] * NTYPES + [wspec, bspec, wspec, bspec],
        out_specs=pl.BlockSpec((1, D), lambda i: (0, 0)),
        out_shape=jax.ShapeDtypeStruct((1, D), jnp.float32),
        scratch_shapes=[pltpu.VMEM((8, D), jnp.float32)],
        compiler_params=pltpu.CompilerParams(
            dimension_semantics=("arbitrary",)),
    )(agg, cnt0, cnt1, z, bias, *h1s, lin2_w, lin2_b, lin_w, lin_b)


def _prep_edges(eis):
    srcs, dsts, flats = [], [], []
    pad = EP - E
    padv = jnp.full((pad,), NP - 1, jnp.int32)
    padc = EPC - E
    for r, ei in enumerate(eis):
        ei = ei.astype(jnp.int32)
        srcs.append(jnp.concatenate([ei[0], padv]).reshape(NS, G, GW))
        dsts.append(jnp.concatenate([ei[1], padv]).reshape(NS, G, GW))
        flat = ei[1] * CW + (r % 6)
        flatpad = jnp.full((padc,), (NP - 1) * CW + (r % 6), jnp.int32)
        flats.append(jnp.concatenate([flat, flatpad]).reshape(NS, GC, 128))
    return jnp.stack(srcs), jnp.stack(dsts), jnp.stack(flats)


def _layer_weights(W_l, b_l, W_r, l):
    wy = W_l[l]
    wr_eff = jnp.stack(
        [jnp.sum(W_r[l, 6:12], axis=0)] + [W_r[l, j] for j in range(6)])
    bias = jnp.stack(
        [jnp.sum(b_l[l, 6:12], axis=0)] + [b_l[l, j] for j in range(6)])
    return wy, wr_eff, bias


def kernel(x_element, x_alignment, x_size, x_element_grouping,
           x_horizontal_grouping, x_vertical_grouping, x_multimodal_grouping,
           edge_index_0, edge_index_1, edge_index_2, edge_index_3,
           edge_index_4, edge_index_5, edge_index_6, edge_index_7,
           edge_index_8, edge_index_9, edge_index_10, edge_index_11,
           W_l, b_l, W_r, lin2_w, lin2_b, lin_w, lin_b, data, batch_size):
    xs = [x_element, x_alignment, x_size, x_element_grouping,
          x_horizontal_grouping, x_vertical_grouping, x_multimodal_grouping]
    xs = [jnp.pad(x.astype(jnp.float32), ((0, NP - N), (0, 0))) for x in xs]
    eis = [edge_index_0, edge_index_1, edge_index_2, edge_index_3,
           edge_index_4, edge_index_5, edge_index_6, edge_index_7,
           edge_index_8, edge_index_9, edge_index_10, edge_index_11]
    src_idx, dst_idx, flat_idx = _prep_edges(eis)

    wy0, wr0, b0 = _layer_weights(W_l, b_l, W_r, 0)
    wy1, wr1, b1 = _layer_weights(W_l, b_l, W_r, 1)

    zeros_tile = jnp.zeros((ROWS_PER_TILE, D), jnp.float32)
    zeros_flat = jnp.zeros((FLATN // NS,), jnp.float32)
    cnt01 = _sc_count(flat_idx, zeros_flat)
    cnt0 = cnt01[0].reshape(NP, CW)
    cnt1 = cnt01[1].reshape(NP, CW)

    y0, z0 = _mm_stage(xs, wy0, wr0)
    agg0 = _sc_segment(y0, src_idx, dst_idx, zeros_tile)

    y1, z1, h1s = _fused_cm_stage(agg0, cnt0, cnt1, z0, b0, wy1, wr1)
    agg1 = _sc_segment(y1, src_idx, dst_idx, zeros_tile)

    out = _final(agg1, cnt0, cnt1, z1, b1, h1s, lin2_w, lin2_b.reshape(1, D),
                 lin_w, lin_b.reshape(1, D))
    out = out * jnp.asarray(batch_size, out.dtype)
    return out + jnp.asarray(data, out.dtype)


# speedup vs baseline: 1.3190x; 1.3190x over previous
import functools

import jax
import jax.numpy as jnp
from jax import lax
from jax.experimental import pallas as pl
from jax.experimental.pallas import tpu as pltpu
from jax.experimental.pallas import tpu_sc as plsc

N = 10000
D = 128
E = 50000
NTYPES = 7
NREL = 12
NP = 10240
NC = 2
NS = 16
G = 49
GW = 64
EP = NS * G * GW
BLK = 1024
NBLK = NP // BLK

SRC_T = [0, 0, 0, 0, 0, 0, 1, 2, 3, 4, 5, 6]
DST_T = [1, 2, 3, 4, 5, 6, 0, 0, 0, 0, 0, 0]


def _mm_body(x0, x1, x2, x3, x4, x5, x6, wy_ref, wr_ref, y_ref, z_ref):
    xs = [x0[...], x1[...], x2[...], x3[...], x4[...], x5[...], x6[...]]
    for r in range(NREL):
        y_ref[r] = jnp.dot(xs[SRC_T[r]], wy_ref[r],
                           preferred_element_type=jnp.float32)
    for t in range(NTYPES):
        z_ref[t] = jnp.dot(xs[t], wr_ref[t], preferred_element_type=jnp.float32)


def _mm_stage(xs7, wy, wr_eff):
    blk_x = pl.BlockSpec((BLK, D), lambda i: (i, 0))
    return pl.pallas_call(
        _mm_body,
        grid=(NBLK,),
        in_specs=[blk_x] * 7 + [
            pl.BlockSpec((NREL, D, D), lambda i: (0, 0, 0)),
            pl.BlockSpec((NTYPES, D, D), lambda i: (0, 0, 0)),
        ],
        out_specs=[
            pl.BlockSpec((NREL, BLK, D), lambda i: (0, i, 0)),
            pl.BlockSpec((NTYPES, BLK, D), lambda i: (0, i, 0)),
        ],
        out_shape=[
            jax.ShapeDtypeStruct((NREL, NP, D), jnp.float32),
            jax.ShapeDtypeStruct((NTYPES, NP, D), jnp.float32),
        ],
    )(*xs7, wy, wr_eff)


ROWS_PER_TILE = NP // NS


NB = 4
AHEAD = 3


def _sc_body(y_hbm, src_hbm, dst_hbm, zeros_hbm, agg_hbm, acc, srcv, dstv,
             rows, zsem, *gsem):
    c = lax.axis_index("c")
    s = lax.axis_index("s")

    def one_rel(rr, carry):
        r = c * 6 + rr
        zd = pltpu.async_copy(
            zeros_hbm, acc.at[pl.ds(s * ROWS_PER_TILE, ROWS_PER_TILE)], zsem)
        pltpu.sync_copy(src_hbm.at[r, s], srcv)
        pltpu.sync_copy(dst_hbm.at[r, s], dstv)
        gd = [None] * G
        for g in range(AHEAD):
            gd[g] = pltpu.async_copy(y_hbm.at[r].at[srcv.at[g]],
                                     rows.at[g % NB], gsem[g % NB])
        zd.wait()
        plsc.subcore_barrier()
        for g in range(G):
            b = g % NB
            gd[g].wait()
            gn = g + AHEAD
            if gn < G:
                gd[gn] = pltpu.async_copy(y_hbm.at[r].at[srcv.at[gn]],
                                          rows.at[gn % NB], gsem[gn % NB])
            pltpu.sync_copy(rows.at[b], acc.at[dstv.at[g]], add=True)
        plsc.subcore_barrier()
        pltpu.sync_copy(acc.at[pl.ds(s * ROWS_PER_TILE, ROWS_PER_TILE)],
                        agg_hbm.at[r, pl.ds(s * ROWS_PER_TILE, ROWS_PER_TILE)])
        return carry

    lax.fori_loop(0, 6, one_rel, 0)


def _sc_segment(y, src_idx, dst_idx, zeros_tile):
    mesh = plsc.VectorSubcoreMesh(core_axis_name="c", subcore_axis_name="s")
    f = pl.kernel(
        _sc_body,
        out_type=jax.ShapeDtypeStruct((NREL, NP, D), jnp.float32),
        mesh=mesh,
        scratch_types=[
            pltpu.VMEM_SHARED((NP, D), jnp.float32),
            pltpu.VMEM((G, GW), jnp.int32),
            pltpu.VMEM((G, GW), jnp.int32),
            pltpu.VMEM((NB, GW, D), jnp.float32),
        ] + [pltpu.SemaphoreType.DMA] * (NB + 1),
    )
    return f(y, src_idx, dst_idx, zeros_tile)


GC = 25
EPC = NS * GC * 128
CW = 8
FLATN = NP * CW


def _sc_count_body(flat_hbm, zeros_hbm, cnt_hbm, acc, flatv, onesv, zsem):
    c = lax.axis_index("c")
    s = lax.axis_index("s")
    one16 = jnp.ones((16,), jnp.float32)
    for j in range(128 // 16):
        onesv[pl.ds(j * 16, 16)] = one16
    chunk = FLATN // NS
    zd = pltpu.async_copy(zeros_hbm, acc.at[pl.ds(s * chunk, chunk)], zsem)

    def one_rel(rr, carry):
        r = c * 6 + rr
        pltpu.sync_copy(flat_hbm.at[r, s], flatv)
        for g in range(GC):
            pltpu.sync_copy(onesv, acc.at[flatv.at[g]], add=True)
        return carry

    pltpu.sync_copy(flat_hbm.at[c * 6, s], flatv)
    zd.wait()
    plsc.subcore_barrier()
    for g in range(GC):
        pltpu.sync_copy(onesv, acc.at[flatv.at[g]], add=True)
    lax.fori_loop(1, 6, one_rel, 0)
    plsc.subcore_barrier()
    pltpu.sync_copy(acc.at[pl.ds(s * chunk, chunk)],
                    cnt_hbm.at[c, pl.ds(s * chunk, chunk)])


def _sc_count(flat_idx, zeros_flat):
    mesh = plsc.VectorSubcoreMesh(core_axis_name="c", subcore_axis_name="s")
    f = pl.kernel(
        _sc_count_body,
        out_type=jax.ShapeDtypeStruct((NC, FLATN), jnp.float32),
        mesh=mesh,
        scratch_types=[
            pltpu.VMEM_SHARED((FLATN,), jnp.float32),
            pltpu.VMEM((GC, 128), jnp.int32),
            pltpu.VMEM((128,), jnp.float32),
            pltpu.SemaphoreType.DMA,
        ],
    )
    return f(flat_idx, zeros_flat)


def _mean_term(agg_ref, cnt0_ref, cnt1_ref, r):
    cref = cnt0_ref if r < 6 else cnt1_ref
    col = r % 6
    c = jnp.maximum(cref[:, col:col + 1], 1.0)
    return agg_ref[r] / c


FBLK = 512


def _fused_cm_body(agg_ref, cnt0_ref, cnt1_ref, z_ref, b_ref, wy_ref, wr_ref,
                  y_ref, zo_ref, *h_ref):
    i = pl.program_id(0)
    rowid = i * FBLK + lax.broadcasted_iota(jnp.int32, (FBLK, 1), 0)
    mask = (rowid < N).astype(jnp.float32)
    hs = []
    for t in range(NTYPES):
        if t == 0:
            m = _mean_term(agg_ref, cnt0_ref, cnt1_ref, 6)
            for r in range(7, 12):
                m = m + _mean_term(agg_ref, cnt0_ref, cnt1_ref, r)
        else:
            m = _mean_term(agg_ref, cnt0_ref, cnt1_ref, t - 1)
        h = jnp.maximum(m + z_ref[t] + b_ref[t:t + 1, :], 0.0) * mask
        h_ref[t][...] = h
        hs.append(h)
    for r in range(NREL):
        y_ref[r] = jnp.dot(hs[SRC_T[r]], wy_ref[r],
                           preferred_element_type=jnp.float32)
    for t in range(NTYPES):
        zo_ref[t] = jnp.dot(hs[t], wr_ref[t], preferred_element_type=jnp.float32)


def _fused_cm_stage(agg, cnt0, cnt1, z, bias, wy, wr_eff):
    blk_h = pl.BlockSpec((FBLK, D), lambda i: (i, 0))
    blk_rel = pl.BlockSpec((NREL, FBLK, D), lambda i: (0, i, 0))
    blk_typ = pl.BlockSpec((NTYPES, FBLK, D), lambda i: (0, i, 0))
    blk_cnt = pl.BlockSpec((FBLK, CW), lambda i: (i, 0))
    outs = pl.pallas_call(
        _fused_cm_body,
        grid=(NP // FBLK,),
        in_specs=[
            blk_rel, blk_cnt, blk_cnt, blk_typ,
            pl.BlockSpec((NTYPES, D), lambda i: (0, 0)),
            pl.BlockSpec((NREL, D, D), lambda i: (0, 0, 0)),
            pl.BlockSpec((NTYPES, D, D), lambda i: (0, 0, 0)),
        ],
        out_specs=[blk_rel, blk_typ] + [blk_h] * NTYPES,
        out_shape=[
            jax.ShapeDtypeStruct((NREL, NP, D), jnp.float32),
            jax.ShapeDtypeStruct((NTYPES, NP, D), jnp.float32),
        ] + [jax.ShapeDtypeStruct((NP, D), jnp.float32)] * NTYPES,
    )(agg, cnt0, cnt1, z, bias, wy, wr_eff)
    return outs[0], outs[1], outs[2:]


def _combine_body(residual, *refs):
    agg_ref, cnt_ref, z_ref, b_ref = refs[:4]
    h1 = refs[4:4 + NTYPES] if residual else ()
    outs = refs[4 + NTYPES:] if residual else refs[4:]
    i = pl.program_id(0)
    rowid = i * BLK + lax.broadcasted_iota(jnp.int32, (BLK, 1), 0)
    mask = (rowid < N).astype(jnp.float32)
    for t in range(NTYPES):
        if t == 0:
            m = _mean_term(agg_ref, cnt_ref, 6)
            for r in range(7, 12):
                m = m + _mean_term(agg_ref, cnt_ref, r)
        else:
            m = _mean_term(agg_ref, cnt_ref, t - 1)
        h = jnp.maximum(m + z_ref[t] + b_ref[t:t + 1, :], 0.0) * mask
        if residual:
            h = h + h1[t][...]
        outs[t][...] = h


def _combine(agg, cnt, z, bias, h1s=None):
    residual = h1s is not None
    blk_out = pl.BlockSpec((BLK, D), lambda i: (i, 0))
    blk_rel = pl.BlockSpec((NREL, BLK, D), lambda i: (0, i, 0))
    in_specs = [
        blk_rel,
        blk_rel,
        pl.BlockSpec((NTYPES, BLK, D), lambda i: (0, i, 0)),
        pl.BlockSpec((NTYPES, D), lambda i: (0, 0)),
    ]
    args = [agg, cnt, z, bias]
    if residual:
        in_specs += [blk_out] * NTYPES
        args += list(h1s)
    return pl.pallas_call(
        functools.partial(_combine_body, residual),
        grid=(NBLK,),
        in_specs=in_specs,
        out_specs=[blk_out] * NTYPES,
        out_shape=[jax.ShapeDtypeStruct((NP, D), jnp.float32)] * NTYPES,
    )(*args)


def _final_body(agg_ref, cnt0_ref, cnt1_ref, z_ref, b_ref, h1_0, h1_1, h1_2,
                h1_3, h1_4, h1_5, h1_6, l2w_ref, l2b_ref, lw_ref, lb_ref,
                out_ref, acc):
    h1 = [h1_0, h1_1, h1_2, h1_3, h1_4, h1_5, h1_6]
    i = pl.program_id(0)

    @pl.when(i == 0)
    def _():
        acc[...] = jnp.zeros((8, D), jnp.float32)

    rowid = i * BLK + lax.broadcasted_iota(jnp.int32, (BLK, 1), 0)
    mask = (rowid < N).astype(jnp.float32)
    psum = jnp.zeros((1, D), jnp.float32)
    for t in range(NTYPES):
        if t == 0:
            m = _mean_term(agg_ref, cnt0_ref, cnt1_ref, 6)
            for r in range(7, 12):
                m = m + _mean_term(agg_ref, cnt0_ref, cnt1_ref, r)
        else:
            m = _mean_term(agg_ref, cnt0_ref, cnt1_ref, t - 1)
        h = jnp.maximum(m + z_ref[t] + b_ref[t:t + 1, :], 0.0) * mask
        h = h + h1[t][...] * mask
        psum = psum + jnp.sum(h, axis=0, keepdims=True)
    acc[0:1, :] += psum

    @pl.when(i == NBLK - 1)
    def _():
        mvec = acc[0:1, :] * (1.0 / (NTYPES * N))
        o = jnp.dot(mvec, l2w_ref[...], preferred_element_type=jnp.float32)
        o = o + l2b_ref[...]
        o = jnp.dot(o, lw_ref[...], preferred_element_type=jnp.float32)
        out_ref[...] = o + lb_ref[...]


def _final(agg, cnt0, cnt1, z, bias, h1s, lin2_w, lin2_b, lin_w, lin_b):
    blk_h = pl.BlockSpec((BLK, D), lambda i: (i, 0))
    blk_rel = pl.BlockSpec((NREL, BLK, D), lambda i: (0, i, 0))
    blk_cnt = pl.BlockSpec((BLK, CW), lambda i: (i, 0))
    wspec = pl.BlockSpec((D, D), lambda i: (0, 0))
    bspec = pl.BlockSpec((1, D), lambda i: (0, 0))
    return pl.pallas_call(
        _final_body,
        grid=(NBLK,),
        in_specs=[
            blk_rel,
            blk_cnt,
            blk_cnt,
            pl.BlockSpec((NTYPES, BLK, D), lambda i: (0, i, 0)),
            pl.BlockSpec((NTYPES, D), lambda i: (0, 0)),
        ] + [blk_h] * NTYPES + [wspec, bspec, wspec, bspec],
        out_specs=pl.BlockSpec((1, D), lambda i: (0, 0)),
        out_shape=jax.ShapeDtypeStruct((1, D), jnp.float32),
        scratch_shapes=[pltpu.VMEM((8, D), jnp.float32)],
        compiler_params=pltpu.CompilerParams(
            dimension_semantics=("arbitrary",)),
    )(agg, cnt0, cnt1, z, bias, *h1s, lin2_w, lin2_b, lin_w, lin_b)


def _prep_edges(eis):
    srcs, dsts, flats = [], [], []
    pad = EP - E
    padv = jnp.full((pad,), NP - 1, jnp.int32)
    padc = EPC - E
    for r, ei in enumerate(eis):
        ei = ei.astype(jnp.int32)
        srcs.append(jnp.concatenate([ei[0], padv]).reshape(NS, G, GW))
        dsts.append(jnp.concatenate([ei[1], padv]).reshape(NS, G, GW))
        flat = ei[1] * CW + (r % 6)
        flatpad = jnp.full((padc,), (NP - 1) * CW + (r % 6), jnp.int32)
        flats.append(jnp.concatenate([flat, flatpad]).reshape(NS, GC, 128))
    return jnp.stack(srcs), jnp.stack(dsts), jnp.stack(flats)


def _layer_weights(W_l, b_l, W_r, l):
    wy = W_l[l]
    wr_eff = jnp.stack(
        [jnp.sum(W_r[l, 6:12], axis=0)] + [W_r[l, j] for j in range(6)])
    bias = jnp.stack(
        [jnp.sum(b_l[l, 6:12], axis=0)] + [b_l[l, j] for j in range(6)])
    return wy, wr_eff, bias


def kernel(x_element, x_alignment, x_size, x_element_grouping,
           x_horizontal_grouping, x_vertical_grouping, x_multimodal_grouping,
           edge_index_0, edge_index_1, edge_index_2, edge_index_3,
           edge_index_4, edge_index_5, edge_index_6, edge_index_7,
           edge_index_8, edge_index_9, edge_index_10, edge_index_11,
           W_l, b_l, W_r, lin2_w, lin2_b, lin_w, lin_b, data, batch_size):
    xs = [x_element, x_alignment, x_size, x_element_grouping,
          x_horizontal_grouping, x_vertical_grouping, x_multimodal_grouping]
    xs = [jnp.pad(x.astype(jnp.float32), ((0, NP - N), (0, 0))) for x in xs]
    eis = [edge_index_0, edge_index_1, edge_index_2, edge_index_3,
           edge_index_4, edge_index_5, edge_index_6, edge_index_7,
           edge_index_8, edge_index_9, edge_index_10, edge_index_11]
    src_idx, dst_idx, flat_idx = _prep_edges(eis)

    wy0, wr0, b0 = _layer_weights(W_l, b_l, W_r, 0)
    wy1, wr1, b1 = _layer_weights(W_l, b_l, W_r, 1)

    zeros_tile = jnp.zeros((ROWS_PER_TILE, D), jnp.float32)
    zeros_flat = jnp.zeros((FLATN // NS,), jnp.float32)
    cnt01 = _sc_count(flat_idx, zeros_flat)
    cnt0 = cnt01[0].reshape(NP, CW)
    cnt1 = cnt01[1].reshape(NP, CW)

    y0, z0 = _mm_stage(xs, wy0, wr0)
    agg0 = _sc_segment(y0, src_idx, dst_idx, zeros_tile)

    y1, z1, h1s = _fused_cm_stage(agg0, cnt0, cnt1, z0, b0, wy1, wr1)
    agg1 = _sc_segment(y1, src_idx, dst_idx, zeros_tile)

    out = _final(agg1, cnt0, cnt1, z1, b1, h1s, lin2_w, lin2_b.reshape(1, D),
                 lin_w, lin_b.reshape(1, D))
    out = out * jnp.asarray(batch_size, out.dtype)
    return out + jnp.asarray(data, out.dtype)


# speedup vs baseline: 1.3195x; 1.0004x over previous
import functools

import jax
import jax.numpy as jnp
from jax import lax
from jax.experimental import pallas as pl
from jax.experimental.pallas import tpu as pltpu
from jax.experimental.pallas import tpu_sc as plsc

N = 10000
D = 128
E = 50000
NTYPES = 7
NREL = 12
NP = 10240
NC = 2
NS = 16
G = 49
GW = 64
EP = NS * G * GW
BLK = 1024
NBLK = NP // BLK

SRC_T = [0, 0, 0, 0, 0, 0, 1, 2, 3, 4, 5, 6]
DST_T = [1, 2, 3, 4, 5, 6, 0, 0, 0, 0, 0, 0]


def _mm_body(x0, x1, x2, x3, x4, x5, x6, wy_ref, wr_ref, y_ref, z_ref):
    xs = [x0[...], x1[...], x2[...], x3[...], x4[...], x5[...], x6[...]]
    for r in range(NREL):
        y_ref[r] = jnp.dot(xs[SRC_T[r]], wy_ref[r],
                           preferred_element_type=jnp.float32)
    for t in range(NTYPES):
        z_ref[t] = jnp.dot(xs[t], wr_ref[t], preferred_element_type=jnp.float32)


def _mm_stage(xs7, wy, wr_eff):
    blk_x = pl.BlockSpec((BLK, D), lambda i: (i, 0))
    return pl.pallas_call(
        _mm_body,
        grid=(NBLK,),
        in_specs=[blk_x] * 7 + [
            pl.BlockSpec((NREL, D, D), lambda i: (0, 0, 0)),
            pl.BlockSpec((NTYPES, D, D), lambda i: (0, 0, 0)),
        ],
        out_specs=[
            pl.BlockSpec((NREL, BLK, D), lambda i: (0, i, 0)),
            pl.BlockSpec((NTYPES, BLK, D), lambda i: (0, i, 0)),
        ],
        out_shape=[
            jax.ShapeDtypeStruct((NREL, NP, D), jnp.float32),
            jax.ShapeDtypeStruct((NTYPES, NP, D), jnp.float32),
        ],
    )(*xs7, wy, wr_eff)


ROWS_PER_TILE = NP // NS


NB = 4
AHEAD = 3


def _sc_body(y_hbm, src_hbm, dst_hbm, zeros_hbm, agg_hbm, acc, srcv, dstv,
             rows, zsem, *gsem):
    c = lax.axis_index("c")
    s = lax.axis_index("s")

    def one_rel(rr, carry):
        r = c * 6 + rr
        zd = pltpu.async_copy(
            zeros_hbm, acc.at[pl.ds(s * ROWS_PER_TILE, ROWS_PER_TILE)], zsem)
        pltpu.sync_copy(src_hbm.at[r, s], srcv)
        pltpu.sync_copy(dst_hbm.at[r, s], dstv)
        gd = [None] * G
        for g in range(AHEAD):
            gd[g] = pltpu.async_copy(y_hbm.at[r].at[srcv.at[g]],
                                     rows.at[g % NB], gsem[g % NB])
        zd.wait()
        plsc.subcore_barrier()
        for g in range(G):
            b = g % NB
            gd[g].wait()
            gn = g + AHEAD
            if gn < G:
                gd[gn] = pltpu.async_copy(y_hbm.at[r].at[srcv.at[gn]],
                                          rows.at[gn % NB], gsem[gn % NB])
            pltpu.sync_copy(rows.at[b], acc.at[dstv.at[g]], add=True)
        plsc.subcore_barrier()
        pltpu.sync_copy(acc.at[pl.ds(s * ROWS_PER_TILE, ROWS_PER_TILE)],
                        agg_hbm.at[r, pl.ds(s * ROWS_PER_TILE, ROWS_PER_TILE)])
        return carry

    lax.fori_loop(0, 6, one_rel, 0)


def _sc_segment(y, src_idx, dst_idx, zeros_tile):
    mesh = plsc.VectorSubcoreMesh(core_axis_name="c", subcore_axis_name="s")
    f = pl.kernel(
        _sc_body,
        out_type=jax.ShapeDtypeStruct((NREL, NP, D), jnp.float32),
        mesh=mesh,
        scratch_types=[
            pltpu.VMEM_SHARED((NP, D), jnp.float32),
            pltpu.VMEM((G, GW), jnp.int32),
            pltpu.VMEM((G, GW), jnp.int32),
            pltpu.VMEM((NB, GW, D), jnp.float32),
        ] + [pltpu.SemaphoreType.DMA] * (NB + 1),
    )
    return f(y, src_idx, dst_idx, zeros_tile)


GC = 25
EPC = NS * GC * 128
CW = 8
FLATN = NP * CW


def _sc_count_body(flat_hbm, zeros_hbm, cnt_hbm, acc, flatv, onesv, zsem):
    c = lax.axis_index("c")
    s = lax.axis_index("s")
    one16 = jnp.ones((16,), jnp.float32)
    for j in range(128 // 16):
        onesv[pl.ds(j * 16, 16)] = one16
    chunk = FLATN // NS
    zd = pltpu.async_copy(zeros_hbm, acc.at[pl.ds(s * chunk, chunk)], zsem)

    def one_rel(rr, carry):
        r = c * 6 + rr
        pltpu.sync_copy(flat_hbm.at[r, s], flatv)
        for g in range(GC):
            pltpu.sync_copy(onesv, acc.at[flatv.at[g]], add=True)
        return carry

    pltpu.sync_copy(flat_hbm.at[c * 6, s], flatv)
    zd.wait()
    plsc.subcore_barrier()
    for g in range(GC):
        pltpu.sync_copy(onesv, acc.at[flatv.at[g]], add=True)
    lax.fori_loop(1, 6, one_rel, 0)
    plsc.subcore_barrier()
    pltpu.sync_copy(acc.at[pl.ds(s * chunk, chunk)],
                    cnt_hbm.at[c, pl.ds(s * chunk, chunk)])


def _sc_count(flat_idx, zeros_flat):
    mesh = plsc.VectorSubcoreMesh(core_axis_name="c", subcore_axis_name="s")
    f = pl.kernel(
        _sc_count_body,
        out_type=jax.ShapeDtypeStruct((NC, FLATN), jnp.float32),
        mesh=mesh,
        scratch_types=[
            pltpu.VMEM_SHARED((FLATN,), jnp.float32),
            pltpu.VMEM((GC, 128), jnp.int32),
            pltpu.VMEM((128,), jnp.float32),
            pltpu.SemaphoreType.DMA,
        ],
    )
    return f(flat_idx, zeros_flat)


def _mean_term(agg_ref, cnt0_ref, cnt1_ref, r):
    cref = cnt0_ref if r < 6 else cnt1_ref
    col = r % 6
    c = jnp.maximum(cref[:, col:col + 1], 1.0)
    return agg_ref[r] / c


FBLK = 1024


def _fused_cm_body(agg_ref, cnt0_ref, cnt1_ref, z_ref, b_ref, wy_ref, wr_ref,
                  y_ref, zo_ref, *h_ref):
    i = pl.program_id(0)
    rowid = i * FBLK + lax.broadcasted_iota(jnp.int32, (FBLK, 1), 0)
    mask = (rowid < N).astype(jnp.float32)
    hs = []
    for t in range(NTYPES):
        if t == 0:
            m = _mean_term(agg_ref, cnt0_ref, cnt1_ref, 6)
            for r in range(7, 12):
                m = m + _mean_term(agg_ref, cnt0_ref, cnt1_ref, r)
        else:
            m = _mean_term(agg_ref, cnt0_ref, cnt1_ref, t - 1)
        h = jnp.maximum(m + z_ref[t] + b_ref[t:t + 1, :], 0.0) * mask
        h_ref[t][...] = h
        hs.append(h)
    for r in range(NREL):
        y_ref[r] = jnp.dot(hs[SRC_T[r]], wy_ref[r],
                           preferred_element_type=jnp.float32)
    for t in range(NTYPES):
        zo_ref[t] = jnp.dot(hs[t], wr_ref[t], preferred_element_type=jnp.float32)


def _fused_cm_stage(agg, cnt0, cnt1, z, bias, wy, wr_eff):
    blk_h = pl.BlockSpec((FBLK, D), lambda i: (i, 0))
    blk_rel = pl.BlockSpec((NREL, FBLK, D), lambda i: (0, i, 0))
    blk_typ = pl.BlockSpec((NTYPES, FBLK, D), lambda i: (0, i, 0))
    blk_cnt = pl.BlockSpec((FBLK, CW), lambda i: (i, 0))
    outs = pl.pallas_call(
        _fused_cm_body,
        grid=(NP // FBLK,),
        in_specs=[
            blk_rel, blk_cnt, blk_cnt, blk_typ,
            pl.BlockSpec((NTYPES, D), lambda i: (0, 0)),
            pl.BlockSpec((NREL, D, D), lambda i: (0, 0, 0)),
            pl.BlockSpec((NTYPES, D, D), lambda i: (0, 0, 0)),
        ],
        out_specs=[blk_rel, blk_typ] + [blk_h] * NTYPES,
        out_shape=[
            jax.ShapeDtypeStruct((NREL, NP, D), jnp.float32),
            jax.ShapeDtypeStruct((NTYPES, NP, D), jnp.float32),
        ] + [jax.ShapeDtypeStruct((NP, D), jnp.float32)] * NTYPES,
    )(agg, cnt0, cnt1, z, bias, wy, wr_eff)
    return outs[0], outs[1], outs[2:]


def _combine_body(residual, *refs):
    agg_ref, cnt_ref, z_ref, b_ref = refs[:4]
    h1 = refs[4:4 + NTYPES] if residual else ()
    outs = refs[4 + NTYPES:] if residual else refs[4:]
    i = pl.program_id(0)
    rowid = i * BLK + lax.broadcasted_iota(jnp.int32, (BLK, 1), 0)
    mask = (rowid < N).astype(jnp.float32)
    for t in range(NTYPES):
        if t == 0:
            m = _mean_term(agg_ref, cnt_ref, 6)
            for r in range(7, 12):
                m = m + _mean_term(agg_ref, cnt_ref, r)
        else:
            m = _mean_term(agg_ref, cnt_ref, t - 1)
        h = jnp.maximum(m + z_ref[t] + b_ref[t:t + 1, :], 0.0) * mask
        if residual:
            h = h + h1[t][...]
        outs[t][...] = h


def _combine(agg, cnt, z, bias, h1s=None):
    residual = h1s is not None
    blk_out = pl.BlockSpec((BLK, D), lambda i: (i, 0))
    blk_rel = pl.BlockSpec((NREL, BLK, D), lambda i: (0, i, 0))
    in_specs = [
        blk_rel,
        blk_rel,
        pl.BlockSpec((NTYPES, BLK, D), lambda i: (0, i, 0)),
        pl.BlockSpec((NTYPES, D), lambda i: (0, 0)),
    ]
    args = [agg, cnt, z, bias]
    if residual:
        in_specs += [blk_out] * NTYPES
        args += list(h1s)
    return pl.pallas_call(
        functools.partial(_combine_body, residual),
        grid=(NBLK,),
        in_specs=in_specs,
        out_specs=[blk_out] * NTYPES,
        out_shape=[jax.ShapeDtypeStruct((NP, D), jnp.float32)] * NTYPES,
    )(*args)


def _final_body(agg_ref, cnt0_ref, cnt1_ref, z_ref, b_ref, h1_0, h1_1, h1_2,
                h1_3, h1_4, h1_5, h1_6, l2w_ref, l2b_ref, lw_ref, lb_ref,
                out_ref, acc):
    h1 = [h1_0, h1_1, h1_2, h1_3, h1_4, h1_5, h1_6]
    i = pl.program_id(0)

    @pl.when(i == 0)
    def _():
        acc[...] = jnp.zeros((8, D), jnp.float32)

    rowid = i * BLK + lax.broadcasted_iota(jnp.int32, (BLK, 1), 0)
    mask = (rowid < N).astype(jnp.float32)
    psum = jnp.zeros((1, D), jnp.float32)
    for t in range(NTYPES):
        if t == 0:
            m = _mean_term(agg_ref, cnt0_ref, cnt1_ref, 6)
            for r in range(7, 12):
                m = m + _mean_term(agg_ref, cnt0_ref, cnt1_ref, r)
        else:
            m = _mean_term(agg_ref, cnt0_ref, cnt1_ref, t - 1)
        h = jnp.maximum(m + z_ref[t] + b_ref[t:t + 1, :], 0.0) * mask
        h = h + h1[t][...] * mask
        psum = psum + jnp.sum(h, axis=0, keepdims=True)
    acc[0:1, :] += psum

    @pl.when(i == NBLK - 1)
    def _():
        mvec = acc[0:1, :] * (1.0 / (NTYPES * N))
        o = jnp.dot(mvec, l2w_ref[...], preferred_element_type=jnp.float32)
        o = o + l2b_ref[...]
        o = jnp.dot(o, lw_ref[...], preferred_element_type=jnp.float32)
        out_ref[...] = o + lb_ref[...]


def _final(agg, cnt0, cnt1, z, bias, h1s, lin2_w, lin2_b, lin_w, lin_b):
    blk_h = pl.BlockSpec((BLK, D), lambda i: (i, 0))
    blk_rel = pl.BlockSpec((NREL, BLK, D), lambda i: (0, i, 0))
    blk_cnt = pl.BlockSpec((BLK, CW), lambda i: (i, 0))
    wspec = pl.BlockSpec((D, D), lambda i: (0, 0))
    bspec = pl.BlockSpec((1, D), lambda i: (0, 0))
    return pl.pallas_call(
        _final_body,
        grid=(NBLK,),
        in_specs=[
            blk_rel,
            blk_cnt,
            blk_cnt,
            pl.BlockSpec((NTYPES, BLK, D), lambda i: (0, i, 0)),
            pl.BlockSpec((NTYPES, D), lambda i: (0, 0)),
        ] + [blk_h] * NTYPES + [wspec, bspec, wspec, bspec],
        out_specs=pl.BlockSpec((1, D), lambda i: (0, 0)),
        out_shape=jax.ShapeDtypeStruct((1, D), jnp.float32),
        scratch_shapes=[pltpu.VMEM((8, D), jnp.float32)],
        compiler_params=pltpu.CompilerParams(
            dimension_semantics=("arbitrary",)),
    )(agg, cnt0, cnt1, z, bias, *h1s, lin2_w, lin2_b, lin_w, lin_b)


def _prep_edges(eis):
    srcs, dsts, flats = [], [], []
    pad = EP - E
    padv = jnp.full((pad,), NP - 1, jnp.int32)
    padc = EPC - E
    for r, ei in enumerate(eis):
        ei = ei.astype(jnp.int32)
        srcs.append(jnp.concatenate([ei[0], padv]).reshape(NS, G, GW))
        dsts.append(jnp.concatenate([ei[1], padv]).reshape(NS, G, GW))
        flat = ei[1] * CW + (r % 6)
        flatpad = jnp.full((padc,), (NP - 1) * CW + (r % 6), jnp.int32)
        flats.append(jnp.concatenate([flat, flatpad]).reshape(NS, GC, 128))
    return jnp.stack(srcs), jnp.stack(dsts), jnp.stack(flats)


def _layer_weights(W_l, b_l, W_r, l):
    wy = W_l[l]
    wr_eff = jnp.stack(
        [jnp.sum(W_r[l, 6:12], axis=0)] + [W_r[l, j] for j in range(6)])
    bias = jnp.stack(
        [jnp.sum(b_l[l, 6:12], axis=0)] + [b_l[l, j] for j in range(6)])
    return wy, wr_eff, bias


def kernel(x_element, x_alignment, x_size, x_element_grouping,
           x_horizontal_grouping, x_vertical_grouping, x_multimodal_grouping,
           edge_index_0, edge_index_1, edge_index_2, edge_index_3,
           edge_index_4, edge_index_5, edge_index_6, edge_index_7,
           edge_index_8, edge_index_9, edge_index_10, edge_index_11,
           W_l, b_l, W_r, lin2_w, lin2_b, lin_w, lin_b, data, batch_size):
    xs = [x_element, x_alignment, x_size, x_element_grouping,
          x_horizontal_grouping, x_vertical_grouping, x_multimodal_grouping]
    xs = [jnp.pad(x.astype(jnp.float32), ((0, NP - N), (0, 0))) for x in xs]
    eis = [edge_index_0, edge_index_1, edge_index_2, edge_index_3,
           edge_index_4, edge_index_5, edge_index_6, edge_index_7,
           edge_index_8, edge_index_9, edge_index_10, edge_index_11]
    src_idx, dst_idx, flat_idx = _prep_edges(eis)

    wy0, wr0, b0 = _layer_weights(W_l, b_l, W_r, 0)
    wy1, wr1, b1 = _layer_weights(W_l, b_l, W_r, 1)

    zeros_tile = jnp.zeros((ROWS_PER_TILE, D), jnp.float32)
    zeros_flat = jnp.zeros((FLATN // NS,), jnp.float32)
    cnt01 = _sc_count(flat_idx, zeros_flat)
    cnt0 = cnt01[0].reshape(NP, CW)
    cnt1 = cnt01[1].reshape(NP, CW)

    y0, z0 = _mm_stage(xs, wy0, wr0)
    agg0 = _sc_segment(y0, src_idx, dst_idx, zeros_tile)

    y1, z1, h1s = _fused_cm_stage(agg0, cnt0, cnt1, z0, b0, wy1, wr1)
    agg1 = _sc_segment(y1, src_idx, dst_idx, zeros_tile)

    out = _final(agg1, cnt0, cnt1, z1, b1, h1s, lin2_w, lin2_b.reshape(1, D),
                 lin_w, lin_b.reshape(1, D))
    out = out * jnp.asarray(batch_size, out.dtype)
    return out + jnp.asarray(data, out.dtype)
